# baseline (speedup 1.0000x reference)
import functools

import jax
import jax.numpy as jnp
from jax import lax
from jax.experimental import pallas as pl
from jax.experimental.pallas import tpu as pltpu
from jax.experimental.pallas import tpu_sc as plsc

N1 = 20000
N2 = 2048
D = 128
DW = 144
LANES = 16
NC, NS = 2, 16
NW = NC * NS
CH = 128
ROWS_PER_TILE = 136
ACC_ROWS = 2176
TRASH = N2


def _make_agg(ept, remap):
    nchunk = ept // CH
    mesh = plsc.VectorSubcoreMesh(core_axis_name="c", subcore_axis_name="s")

    @functools.partial(
        pl.kernel,
        out_type=jax.ShapeDtypeStruct((NC, ACC_ROWS, DW), jnp.float32),
        mesh=mesh,
        compiler_params=pltpu.CompilerParams(
            use_tc_tiling_on_sc=False, needs_layout_passes=False),
        scratch_types=(
            pltpu.VMEM((ept,), jnp.int32),
            pltpu.VMEM((ept,), jnp.int32),
            pltpu.VMEM((ept + CH,), jnp.int32),
            pltpu.VMEM((ept + CH,), jnp.int32),
            pltpu.VMEM((CH,), jnp.int32),
            pltpu.VMEM((CH,), jnp.int32),
            pltpu.VMEM((CH, DW), jnp.float32),
            pltpu.VMEM((ROWS_PER_TILE, DW), jnp.float32),
            pltpu.VMEM_SHARED((ACC_ROWS, DW), jnp.float32),
            pltpu.SemaphoreType.DMA,
        ),
    )
    def agg(table, srcs, dsts, zrow, out_acc,
            srcf, dstf, srcc, dstc, srcix, dstix, rows, stage, acc, gsem):
        c = lax.axis_index("c")
        s = lax.axis_index("s")
        w = s * NC + c
        rbase = s * ROWS_PER_TILE

        pltpu.sync_copy(zrow.at[pl.ds(rbase, ROWS_PER_TILE)], stage)
        pltpu.sync_copy(stage, acc.at[pl.ds(rbase, ROWS_PER_TILE)])

        pltpu.sync_copy(srcs.at[w], srcf.at[pl.ds(0, ept)])
        pltpu.sync_copy(dsts.at[w], dstf.at[pl.ds(0, ept)])

        if remap:
            lane = lax.iota(jnp.int32, LANES)
            unroll = 8
            assert (ept // LANES) % unroll == 0

            def cpbody(i, curv):
                for u in range(unroll):
                    g = i * unroll + u
                    dv = dstf[pl.ds(g * LANES, LANES)]
                    sv = srcf[pl.ds(g * LANES, LANES)]
                    m = dv < N2
                    mi = jnp.where(m, 1, 0)
                    incl = plsc.cumsum(mi)
                    pos = jnp.where(m, curv + incl - mi, ept + lane)
                    plsc.store_scatter(dstc, [pos], dv)
                    plsc.store_scatter(srcc, [pos], sv)
                    curv = curv + plsc.all_reduce_population_count(m)
                return curv
            curv = lax.fori_loop(
                0, ept // LANES // unroll, cpbody,
                jnp.zeros((LANES,), jnp.int32))
            k = curv[0]
            for t in range(CH // LANES):
                srcc[pl.ds(k + t * LANES, LANES)] = jnp.zeros(
                    (LANES,), jnp.int32)
                dstc[pl.ds(k + t * LANES, LANES)] = jnp.full(
                    (LANES,), TRASH, jnp.int32)
            nch = (k + CH - 1) // CH
            sbuf, dbuf = srcc, dstc
        else:
            nch = nchunk
            sbuf, dbuf = srcf, dstf

        plsc.subcore_barrier()

        def cbody(j, carry):
            for t in range(CH // LANES):
                srcix[pl.ds(t * LANES, LANES)] = (
                    sbuf[pl.ds(j * CH + t * LANES, LANES)])
                dstix[pl.ds(t * LANES, LANES)] = (
                    dbuf[pl.ds(j * CH + t * LANES, LANES)])
            pltpu.async_copy(table.at[srcix], rows, gsem).wait()
            pltpu.sync_copy(rows, acc.at[dstix], add=True)
            return carry
        lax.fori_loop(0, nch, cbody, 0)

        plsc.subcore_barrier()

        pltpu.sync_copy(acc.at[pl.ds(rbase, ROWS_PER_TILE)], stage)
        pltpu.sync_copy(stage, out_acc.at[c].at[pl.ds(rbase, ROWS_PER_TILE)])

    return agg


_aggs = {}


def _get_agg(ept, remap):
    key = (ept, remap)
    if key not in _aggs:
        _aggs[key] = _make_agg(ept, remap)
    return _aggs[key]


def _dense_body(accp, xt, wl, bl, wr, out, *, sig, ones_col):
    a = accp[0, :N2, :D] + accp[1, :N2, :D]
    ct = accp[0, :N2, D] + accp[1, :N2, D]
    inv = 1.0 / jnp.maximum(ct, 1.0)
    mean = a * inv[:, None]
    r = (jnp.dot(mean, wl[...], preferred_element_type=jnp.float32)
         + jnp.dot(xt[...], wr[...], preferred_element_type=jnp.float32)
         + bl[...])
    if sig:
        r = 1.0 / (1.0 + jnp.exp(-r))
    if ones_col:
        out[:, :D] = r
        out[:, D:] = jnp.ones((N2, DW - D), jnp.float32)
    else:
        out[...] = r


def _dense(accp, xt, wl, bl, wr, sig, ones_col):
    width = DW if ones_col else D
    return pl.pallas_call(
        functools.partial(_dense_body, sig=sig, ones_col=ones_col),
        out_shape=jax.ShapeDtypeStruct((N2, width), jnp.float32),
    )(accp, xt, wl, bl, wr)


def kernel(x, edge_index1, edge_index2, size1_dst, size2_dst,
           W_l1, b_l1, W_r1, W_l2, b_l2, W_r2):
    e1 = edge_index1.shape[1]
    e2 = edge_index2.shape[1]
    ept1 = e1 // NW
    ept1p = -(-ept1 // CH) * CH
    ept2 = e2 // NW

    agg1 = _get_agg(ept1p, True)
    agg2 = _get_agg(ept2, False)

    pad1 = ept1p - ept1
    src1 = jnp.pad(edge_index1[0].reshape(NW, ept1), ((0, 0), (0, pad1)))
    dst1 = jnp.pad(edge_index1[1].reshape(NW, ept1), ((0, 0), (0, pad1)),
                   constant_values=TRASH)
    src2 = edge_index2[0].reshape(NW, ept2)
    dst2 = edge_index2[1].reshape(NW, ept2)

    zrow = jnp.zeros((ACC_ROWS, DW), jnp.float32)
    xa = jnp.concatenate(
        [x[:N1], jnp.ones((N1, DW - D), jnp.float32)], axis=1)

    accp1 = agg1(xa, src1, dst1, zrow)
    ha = _dense(accp1, x[:N2], W_l1, b_l1.reshape(1, D), W_r1, True, True)
    accp2 = agg2(ha, src2, dst2, zrow)
    out = _dense(accp2, ha[:, :D], W_l2, b_l2.reshape(1, D), W_r2,
                 False, False)
    return out


# speedup vs baseline: 1.0717x; 1.0717x over previous
import functools

import jax
import jax.numpy as jnp
from jax import lax
from jax.experimental import pallas as pl
from jax.experimental.pallas import tpu as pltpu
from jax.experimental.pallas import tpu_sc as plsc

N1 = 20000
N2 = 2048
D = 128
DW = 144
LANES = 16
NC, NS = 2, 16
NW = NC * NS
CH = 128
ROWS_PER_TILE = 136
ACC_ROWS = 2176
TRASH = N2


def _make_agg(ept, remap):
    nchunk = ept // CH
    mesh = plsc.VectorSubcoreMesh(core_axis_name="c", subcore_axis_name="s")

    @functools.partial(
        pl.kernel,
        out_type=jax.ShapeDtypeStruct((NC, ACC_ROWS, DW), jnp.float32),
        mesh=mesh,
        compiler_params=pltpu.CompilerParams(
            use_tc_tiling_on_sc=False, needs_layout_passes=False),
        scratch_types=(
            pltpu.VMEM((ept,), jnp.int32),
            pltpu.VMEM((ept,), jnp.int32),
            pltpu.VMEM((ept + CH,), jnp.int32),
            pltpu.VMEM((ept + CH,), jnp.int32),
            pltpu.VMEM((CH,), jnp.int32),
            pltpu.VMEM((CH,), jnp.int32),
            pltpu.VMEM((CH, DW), jnp.float32),
            pltpu.VMEM_SHARED((ACC_ROWS, DW), jnp.float32),
            pltpu.SemaphoreType.DMA,
        ),
    )
    def agg(table, edges, zrow, out_acc,
            srcf, dstf, srcc, dstc, srcix, dstix, rows, acc, gsem):
        c = lax.axis_index("c")
        s = lax.axis_index("s")
        w = s * NC + c
        rbase = s * ROWS_PER_TILE

        pltpu.sync_copy(zrow.at[pl.ds(rbase, ROWS_PER_TILE)],
                        acc.at[pl.ds(rbase, ROWS_PER_TILE)])

        pltpu.sync_copy(edges.at[pl.ds(w * ept, ept)], srcf)
        pltpu.sync_copy(edges.at[pl.ds((NW + w) * ept, ept)], dstf)

        if remap:
            lane = lax.iota(jnp.int32, LANES)
            unroll = next(u for u in (8, 5, 4, 2, 1)
                          if (ept // LANES) % u == 0)

            def cpbody(i, curv):
                for u in range(unroll):
                    g = i * unroll + u
                    dv = dstf[pl.ds(g * LANES, LANES)]
                    sv = srcf[pl.ds(g * LANES, LANES)]
                    m = dv < N2
                    mi = jnp.where(m, 1, 0)
                    incl = plsc.cumsum(mi)
                    pos = jnp.where(m, curv + incl - mi, ept + lane)
                    plsc.store_scatter(dstc, [pos], dv)
                    plsc.store_scatter(srcc, [pos], sv)
                    curv = curv + plsc.all_reduce_population_count(m)
                return curv
            curv = lax.fori_loop(
                0, ept // LANES // unroll, cpbody,
                jnp.zeros((LANES,), jnp.int32))
            k = curv[0]
            for t in range(CH // LANES):
                srcc[pl.ds(k + t * LANES, LANES)] = jnp.zeros(
                    (LANES,), jnp.int32)
                dstc[pl.ds(k + t * LANES, LANES)] = jnp.full(
                    (LANES,), TRASH, jnp.int32)
            nch = (k + CH - 1) // CH
            sbuf, dbuf = srcc, dstc
        else:
            nch = nchunk
            sbuf, dbuf = srcf, dstf

        plsc.subcore_barrier()

        def cbody(j, carry):
            for t in range(CH // LANES):
                srcix[pl.ds(t * LANES, LANES)] = (
                    sbuf[pl.ds(j * CH + t * LANES, LANES)])
                dstix[pl.ds(t * LANES, LANES)] = (
                    dbuf[pl.ds(j * CH + t * LANES, LANES)])
            pltpu.async_copy(table.at[srcix], rows, gsem).wait()
            pltpu.sync_copy(rows, acc.at[dstix], add=True)
            return carry
        lax.fori_loop(0, nch, cbody, 0)

        plsc.subcore_barrier()

        pltpu.sync_copy(acc.at[pl.ds(rbase, ROWS_PER_TILE)],
                        out_acc.at[c].at[pl.ds(rbase, ROWS_PER_TILE)])

    return agg


_aggs = {}


def _get_agg(ept, remap):
    key = (ept, remap)
    if key not in _aggs:
        _aggs[key] = _make_agg(ept, remap)
    return _aggs[key]


def _dense_body(accp, xt, wl, bl, wr, out, *, sig, ones_col):
    a = accp[0, :N2, :D] + accp[1, :N2, :D]
    ct = accp[0, :N2, D] + accp[1, :N2, D]
    inv = 1.0 / jnp.maximum(ct, 1.0)
    mean = a * inv[:, None]
    r = (jnp.dot(mean, wl[...], preferred_element_type=jnp.float32)
         + jnp.dot(xt[...], wr[...], preferred_element_type=jnp.float32)
         + bl[...])
    if sig:
        r = 1.0 / (1.0 + jnp.exp(-r))
    if ones_col:
        out[:, :D] = r
        out[:, D:] = jnp.ones((N2, DW - D), jnp.float32)
    else:
        out[...] = r


def _dense(accp, xt, wl, bl, wr, sig, ones_col):
    width = DW if ones_col else D
    return pl.pallas_call(
        functools.partial(_dense_body, sig=sig, ones_col=ones_col),
        out_shape=jax.ShapeDtypeStruct((N2, width), jnp.float32),
    )(accp, xt, wl, bl, wr)


def kernel(x, edge_index1, edge_index2, size1_dst, size2_dst,
           W_l1, b_l1, W_r1, W_l2, b_l2, W_r2):
    e1 = edge_index1.shape[1]
    e2 = edge_index2.shape[1]

    agg1 = _get_agg(e1 // NW, True)
    agg2 = _get_agg(e2 // NW, False)

    zrow = jnp.zeros((ACC_ROWS, DW), jnp.float32)
    xa = jnp.concatenate(
        [x[:N1], jnp.ones((N1, DW - D), jnp.float32)], axis=1)

    accp1 = agg1(xa, edge_index1.reshape(2 * e1), zrow)
    ha = _dense(accp1, x[:N2], W_l1, b_l1.reshape(1, D), W_r1, True, True)
    accp2 = agg2(ha, edge_index2.reshape(2 * e2), zrow)
    out = _dense(accp2, ha[:, :D], W_l2, b_l2.reshape(1, D), W_r2,
                 False, False)
    return out


# speedup vs baseline: 1.0841x; 1.0116x over previous
import functools

import jax
import jax.numpy as jnp
from jax import lax
from jax.experimental import pallas as pl
from jax.experimental.pallas import tpu as pltpu
from jax.experimental.pallas import tpu_sc as plsc

N1 = 20000
N2 = 2048
D = 128
DW = 144
LANES = 16
NC, NS = 2, 16
NW = NC * NS
CH = 128
ROWS_PER_TILE = 136
ACC_ROWS = 2176
TRASH = N2


def _make_agg(ept, remap):
    nchunk = ept // CH
    mesh = plsc.VectorSubcoreMesh(core_axis_name="c", subcore_axis_name="s")

    @functools.partial(
        pl.kernel,
        out_type=jax.ShapeDtypeStruct((NC, ACC_ROWS, DW), jnp.float32),
        mesh=mesh,
        compiler_params=pltpu.CompilerParams(
            use_tc_tiling_on_sc=False, needs_layout_passes=False),
        scratch_types=(
            pltpu.VMEM((ept,), jnp.int32),
            pltpu.VMEM((ept,), jnp.int32),
            pltpu.VMEM((ept + CH,), jnp.int32),
            pltpu.VMEM((ept + CH,), jnp.int32),
            pltpu.VMEM((CH,), jnp.int32),
            pltpu.VMEM((CH // 2,), jnp.int32),
            pltpu.VMEM((CH // 2,), jnp.int32),
            pltpu.VMEM((CH // 2, DW), jnp.float32),
            pltpu.VMEM((CH // 2, DW), jnp.float32),
            pltpu.VMEM_SHARED((ACC_ROWS, DW), jnp.float32),
            pltpu.SemaphoreType.DMA,
            pltpu.SemaphoreType.DMA,
        ),
    )
    def agg(table, edges, zrow, out_acc,
            srcf, dstf, srcc, dstc, srcix, dstix0, dstix1, rows0, rows1,
            acc, gsem0, gsem1):
        c = lax.axis_index("c")
        s = lax.axis_index("s")
        w = s * NC + c
        rbase = s * ROWS_PER_TILE

        pltpu.sync_copy(zrow.at[pl.ds(rbase, ROWS_PER_TILE)],
                        acc.at[pl.ds(rbase, ROWS_PER_TILE)])

        pltpu.sync_copy(edges.at[pl.ds(w * ept, ept)], srcf)
        pltpu.sync_copy(edges.at[pl.ds((NW + w) * ept, ept)], dstf)

        if remap:
            lane = lax.iota(jnp.int32, LANES)
            unroll = next(u for u in (8, 5, 4, 2, 1)
                          if (ept // LANES) % u == 0)

            def cpbody(i, curv):
                for u in range(unroll):
                    g = i * unroll + u
                    dv = dstf[pl.ds(g * LANES, LANES)]
                    sv = srcf[pl.ds(g * LANES, LANES)]
                    m = dv < N2
                    mi = jnp.where(m, 1, 0)
                    incl = plsc.cumsum(mi)
                    pos = jnp.where(m, curv + incl - mi, ept + lane)
                    plsc.store_scatter(dstc, [pos], dv)
                    plsc.store_scatter(srcc, [pos], sv)
                    curv = curv + plsc.all_reduce_population_count(m)
                return curv
            curv = lax.fori_loop(
                0, ept // LANES // unroll, cpbody,
                jnp.zeros((LANES,), jnp.int32))
            k = curv[0]
            for t in range(CH // LANES):
                srcc[pl.ds(k + t * LANES, LANES)] = jnp.zeros(
                    (LANES,), jnp.int32)
                dstc[pl.ds(k + t * LANES, LANES)] = jnp.full(
                    (LANES,), TRASH, jnp.int32)
            nch = (k + CH - 1) // CH
            sbuf, dbuf = srcc, dstc
        else:
            nch = nchunk
            sbuf, dbuf = srcf, dstf

        plsc.subcore_barrier()

        half = CH // 2

        def cbody(j, carry):
            for t in range(CH // LANES):
                srcix[pl.ds(t * LANES, LANES)] = (
                    sbuf[pl.ds(j * CH + t * LANES, LANES)])
            for t in range(half // LANES):
                dstix0[pl.ds(t * LANES, LANES)] = (
                    dbuf[pl.ds(j * CH + t * LANES, LANES)])
                dstix1[pl.ds(t * LANES, LANES)] = (
                    dbuf[pl.ds(j * CH + half + t * LANES, LANES)])
            c0 = pltpu.async_copy(
                table.at[srcix.at[pl.ds(0, half)]], rows0, gsem0)
            c1 = pltpu.async_copy(
                table.at[srcix.at[pl.ds(half, half)]], rows1, gsem1)
            c0.wait()
            pltpu.sync_copy(rows0, acc.at[dstix0], add=True)
            c1.wait()
            pltpu.sync_copy(rows1, acc.at[dstix1], add=True)
            return carry
        lax.fori_loop(0, nch, cbody, 0)

        plsc.subcore_barrier()

        pltpu.sync_copy(acc.at[pl.ds(rbase, ROWS_PER_TILE)],
                        out_acc.at[c].at[pl.ds(rbase, ROWS_PER_TILE)])

    return agg


_aggs = {}


def _get_agg(ept, remap):
    key = (ept, remap)
    if key not in _aggs:
        _aggs[key] = _make_agg(ept, remap)
    return _aggs[key]


def _dense_body(accp, xt, wl, bl, wr, out, *, sig, ones_col):
    a = accp[0, :N2, :D] + accp[1, :N2, :D]
    ct = accp[0, :N2, D] + accp[1, :N2, D]
    inv = 1.0 / jnp.maximum(ct, 1.0)
    mean = a * inv[:, None]
    r = (jnp.dot(mean, wl[...], preferred_element_type=jnp.float32)
         + jnp.dot(xt[...], wr[...], preferred_element_type=jnp.float32)
         + bl[...])
    if sig:
        r = 1.0 / (1.0 + jnp.exp(-r))
    if ones_col:
        out[:, :D] = r
        out[:, D:] = jnp.ones((N2, DW - D), jnp.float32)
    else:
        out[...] = r


def _dense(accp, xt, wl, bl, wr, sig, ones_col):
    width = DW if ones_col else D
    return pl.pallas_call(
        functools.partial(_dense_body, sig=sig, ones_col=ones_col),
        out_shape=jax.ShapeDtypeStruct((N2, width), jnp.float32),
    )(accp, xt, wl, bl, wr)


def kernel(x, edge_index1, edge_index2, size1_dst, size2_dst,
           W_l1, b_l1, W_r1, W_l2, b_l2, W_r2):
    e1 = edge_index1.shape[1]
    e2 = edge_index2.shape[1]

    agg1 = _get_agg(e1 // NW, True)
    agg2 = _get_agg(e2 // NW, False)

    zrow = jnp.zeros((ACC_ROWS, DW), jnp.float32)
    xa = jnp.concatenate(
        [x[:N1], jnp.ones((N1, DW - D), jnp.float32)], axis=1)

    accp1 = agg1(xa, edge_index1.reshape(2 * e1), zrow)
    ha = _dense(accp1, x[:N2], W_l1, b_l1.reshape(1, D), W_r1, True, True)
    accp2 = agg2(ha, edge_index2.reshape(2 * e2), zrow)
    out = _dense(accp2, ha[:, :D], W_l2, b_l2.reshape(1, D), W_r2,
                 False, False)
    return out


# speedup vs baseline: 1.3055x; 1.2042x over previous
import functools

import jax
import jax.numpy as jnp
from jax import lax
from jax.experimental import pallas as pl
from jax.experimental.pallas import tpu as pltpu
from jax.experimental.pallas import tpu_sc as plsc

N1 = 20000
N2 = 2048
D = 128
CW = 16
LANES = 16
NC, NS = 2, 16
NW = NC * NS
CH = 128
ROWS_PER_TILE = 136
ACC_ROWS = 2176
TRASH = N2


def _make_agg(ept, remap):
    nchunk = ept // CH
    mesh = plsc.VectorSubcoreMesh(core_axis_name="c", subcore_axis_name="s")

    @functools.partial(
        pl.kernel,
        out_type=(jax.ShapeDtypeStruct((NC, ACC_ROWS, D), jnp.float32),
                  jax.ShapeDtypeStruct((NC, ACC_ROWS, CW), jnp.float32)),
        mesh=mesh,
        compiler_params=pltpu.CompilerParams(
            use_tc_tiling_on_sc=False, needs_layout_passes=False),
        scratch_types=(
            pltpu.VMEM((ept,), jnp.int32),
            pltpu.VMEM((ept,), jnp.int32),
            pltpu.VMEM((ept + CH,), jnp.int32),
            pltpu.VMEM((ept + CH,), jnp.int32),
            pltpu.VMEM((CH,), jnp.int32),
            pltpu.VMEM((CH // 2,), jnp.int32),
            pltpu.VMEM((CH // 2,), jnp.int32),
            pltpu.VMEM((CH // 2, D), jnp.float32),
            pltpu.VMEM((CH // 2, D), jnp.float32),
            pltpu.VMEM((CH // 2, CW), jnp.float32),
            pltpu.VMEM_SHARED((ACC_ROWS, D), jnp.float32),
            pltpu.VMEM_SHARED((ACC_ROWS, CW), jnp.float32),
            pltpu.SemaphoreType.DMA,
            pltpu.SemaphoreType.DMA,
        ),
    )
    def agg(table, edges, zrow, zcnt, onesh, out_acc, out_cnt,
            srcf, dstf, srcc, dstc, srcix, dstix0, dstix1, rows0, rows1,
            ones, acc, cnt, gsem0, gsem1):
        c = lax.axis_index("c")
        s = lax.axis_index("s")
        w = s * NC + c
        rbase = s * ROWS_PER_TILE

        pltpu.sync_copy(zrow.at[pl.ds(rbase, ROWS_PER_TILE)],
                        acc.at[pl.ds(rbase, ROWS_PER_TILE)])
        pltpu.sync_copy(zcnt.at[pl.ds(rbase, ROWS_PER_TILE)],
                        cnt.at[pl.ds(rbase, ROWS_PER_TILE)])
        pltpu.sync_copy(onesh, ones)

        pltpu.sync_copy(edges.at[pl.ds(w * ept, ept)], srcf)
        pltpu.sync_copy(edges.at[pl.ds((NW + w) * ept, ept)], dstf)

        if remap:
            lane = lax.iota(jnp.int32, LANES)
            unroll = next(u for u in (8, 5, 4, 2, 1)
                          if (ept // LANES) % u == 0)

            def cpbody(i, curv):
                for u in range(unroll):
                    g = i * unroll + u
                    dv = dstf[pl.ds(g * LANES, LANES)]
                    sv = srcf[pl.ds(g * LANES, LANES)]
                    m = dv < N2
                    mi = jnp.where(m, 1, 0)
                    incl = plsc.cumsum(mi)
                    pos = jnp.where(m, curv + incl - mi, ept + lane)
                    plsc.store_scatter(dstc, [pos], dv)
                    plsc.store_scatter(srcc, [pos], sv)
                    curv = curv + plsc.all_reduce_population_count(m)
                return curv
            curv = lax.fori_loop(
                0, ept // LANES // unroll, cpbody,
                jnp.zeros((LANES,), jnp.int32))
            k = curv[0]
            for t in range(CH // LANES):
                srcc[pl.ds(k + t * LANES, LANES)] = jnp.zeros(
                    (LANES,), jnp.int32)
                dstc[pl.ds(k + t * LANES, LANES)] = jnp.full(
                    (LANES,), TRASH, jnp.int32)
            nch = (k + CH - 1) // CH
            sbuf, dbuf = srcc, dstc
        else:
            nch = nchunk
            sbuf, dbuf = srcf, dstf

        plsc.subcore_barrier()

        half = CH // 2

        def cbody(j, carry):
            for t in range(CH // LANES):
                srcix[pl.ds(t * LANES, LANES)] = (
                    sbuf[pl.ds(j * CH + t * LANES, LANES)])
            for t in range(half // LANES):
                dstix0[pl.ds(t * LANES, LANES)] = (
                    dbuf[pl.ds(j * CH + t * LANES, LANES)])
                dstix1[pl.ds(t * LANES, LANES)] = (
                    dbuf[pl.ds(j * CH + half + t * LANES, LANES)])
            c0 = pltpu.async_copy(
                table.at[srcix.at[pl.ds(0, half)]], rows0, gsem0)
            c1 = pltpu.async_copy(
                table.at[srcix.at[pl.ds(half, half)]], rows1, gsem1)
            c0.wait()
            pltpu.sync_copy(rows0, acc.at[dstix0], add=True)
            pltpu.sync_copy(ones, cnt.at[dstix0], add=True)
            c1.wait()
            pltpu.sync_copy(rows1, acc.at[dstix1], add=True)
            pltpu.sync_copy(ones, cnt.at[dstix1], add=True)
            return carry
        lax.fori_loop(0, nch, cbody, 0)

        plsc.subcore_barrier()

        pltpu.sync_copy(acc.at[pl.ds(rbase, ROWS_PER_TILE)],
                        out_acc.at[c].at[pl.ds(rbase, ROWS_PER_TILE)])
        pltpu.sync_copy(cnt.at[pl.ds(rbase, ROWS_PER_TILE)],
                        out_cnt.at[c].at[pl.ds(rbase, ROWS_PER_TILE)])

    return agg


_aggs = {}


def _get_agg(ept, remap):
    key = (ept, remap)
    if key not in _aggs:
        _aggs[key] = _make_agg(ept, remap)
    return _aggs[key]


def _dense_body(accp, cntp, xt, wl, bl, wr, out, *, sig):
    a = accp[0, :N2, :] + accp[1, :N2, :]
    ct = cntp[0, :N2, 0] + cntp[1, :N2, 0]
    inv = 1.0 / jnp.maximum(ct, 1.0)
    mean = a * inv[:, None]
    r = (jnp.dot(mean, wl[...], preferred_element_type=jnp.float32)
         + jnp.dot(xt[...], wr[...], preferred_element_type=jnp.float32)
         + bl[...])
    if sig:
        r = 1.0 / (1.0 + jnp.exp(-r))
    out[...] = r


def _dense(accp, cntp, xt, wl, bl, wr, sig):
    return pl.pallas_call(
        functools.partial(_dense_body, sig=sig),
        out_shape=jax.ShapeDtypeStruct((N2, D), jnp.float32),
    )(accp, cntp, xt, wl, bl, wr)


def kernel(x, edge_index1, edge_index2, size1_dst, size2_dst,
           W_l1, b_l1, W_r1, W_l2, b_l2, W_r2):
    e1 = edge_index1.shape[1]
    e2 = edge_index2.shape[1]

    agg1 = _get_agg(e1 // NW, True)
    agg2 = _get_agg(e2 // NW, False)

    zrow = jnp.zeros((ACC_ROWS, D), jnp.float32)
    zcnt = jnp.zeros((ACC_ROWS, CW), jnp.float32)
    onesh = jnp.ones((CH // 2, CW), jnp.float32)

    accp1, cntp1 = agg1(x, edge_index1.reshape(2 * e1), zrow, zcnt, onesh)
    h = _dense(accp1, cntp1, x[:N2], W_l1, b_l1.reshape(1, D), W_r1, True)
    accp2, cntp2 = agg2(h, edge_index2.reshape(2 * e2), zrow, zcnt, onesh)
    out = _dense(accp2, cntp2, h, W_l2, b_l2.reshape(1, D), W_r2, False)
    return out


# speedup vs baseline: 1.3345x; 1.0222x over previous
import functools

import jax
import jax.numpy as jnp
from jax import lax
from jax.experimental import pallas as pl
from jax.experimental.pallas import tpu as pltpu
from jax.experimental.pallas import tpu_sc as plsc

N1 = 20000
N2 = 2048
D = 128
CW = 16
LANES = 16
NC, NS = 2, 16
NW = NC * NS
CH = 128
ROWS_PER_TILE = 136
ACC_ROWS = 2176
TRASH = N2


def _make_agg(ept, remap, stage_table=False):
    nchunk = ept // CH
    mesh = plsc.VectorSubcoreMesh(core_axis_name="c", subcore_axis_name="s")

    @functools.partial(
        pl.kernel,
        out_type=(jax.ShapeDtypeStruct((NC, ACC_ROWS, D), jnp.float32),
                  jax.ShapeDtypeStruct((NC, ACC_ROWS, CW), jnp.float32)),
        mesh=mesh,
        compiler_params=pltpu.CompilerParams(
            use_tc_tiling_on_sc=False, needs_layout_passes=False),
        scratch_types=(
            pltpu.VMEM((ept,), jnp.int32),
            pltpu.VMEM((ept,), jnp.int32),
            pltpu.VMEM((ept + CH,), jnp.int32),
            pltpu.VMEM((ept + CH,), jnp.int32),
            pltpu.VMEM((CH,), jnp.int32),
            pltpu.VMEM((CH // 2,), jnp.int32),
            pltpu.VMEM((CH // 2,), jnp.int32),
            pltpu.VMEM((CH // 2, D), jnp.float32),
            pltpu.VMEM((CH // 2, D), jnp.float32),
            pltpu.VMEM((CH // 2, CW), jnp.float32),
            pltpu.VMEM_SHARED((ACC_ROWS, D), jnp.float32),
            pltpu.VMEM_SHARED((ACC_ROWS, CW), jnp.float32),
            pltpu.VMEM_SHARED((N2, D), jnp.float32),
            pltpu.SemaphoreType.DMA,
            pltpu.SemaphoreType.DMA,
        ),
    )
    def agg(table, edges, zrow, zcnt, onesh, out_acc, out_cnt,
            srcf, dstf, srcc, dstc, srcix, dstix0, dstix1, rows0, rows1,
            ones, acc, cnt, tabsh, gsem0, gsem1):
        c = lax.axis_index("c")
        s = lax.axis_index("s")
        w = s * NC + c
        rbase = s * ROWS_PER_TILE

        pltpu.sync_copy(zrow.at[pl.ds(rbase, ROWS_PER_TILE)],
                        acc.at[pl.ds(rbase, ROWS_PER_TILE)])
        pltpu.sync_copy(zcnt.at[pl.ds(rbase, ROWS_PER_TILE)],
                        cnt.at[pl.ds(rbase, ROWS_PER_TILE)])
        pltpu.sync_copy(onesh, ones)

        pltpu.sync_copy(edges.at[pl.ds(w * ept, ept)], srcf)
        pltpu.sync_copy(edges.at[pl.ds((NW + w) * ept, ept)], dstf)

        if stage_table:
            trows = N2 // NS
            pltpu.sync_copy(table.at[pl.ds(s * trows, trows)],
                            tabsh.at[pl.ds(s * trows, trows)])
        tab = tabsh if stage_table else table

        if remap:
            lane = lax.iota(jnp.int32, LANES)
            unroll = next(u for u in (8, 5, 4, 2, 1)
                          if (ept // LANES) % u == 0)

            def cpbody(i, curv):
                for u in range(unroll):
                    g = i * unroll + u
                    dv = dstf[pl.ds(g * LANES, LANES)]
                    sv = srcf[pl.ds(g * LANES, LANES)]
                    m = dv < N2
                    mi = jnp.where(m, 1, 0)
                    incl = plsc.cumsum(mi)
                    pos = jnp.where(m, curv + incl - mi, ept + lane)
                    plsc.store_scatter(dstc, [pos], dv)
                    plsc.store_scatter(srcc, [pos], sv)
                    curv = curv + plsc.all_reduce_population_count(m)
                return curv
            curv = lax.fori_loop(
                0, ept // LANES // unroll, cpbody,
                jnp.zeros((LANES,), jnp.int32))
            k = curv[0]
            for t in range(CH // LANES):
                srcc[pl.ds(k + t * LANES, LANES)] = jnp.zeros(
                    (LANES,), jnp.int32)
                dstc[pl.ds(k + t * LANES, LANES)] = jnp.full(
                    (LANES,), TRASH, jnp.int32)
            nch = (k + CH - 1) // CH
            sbuf, dbuf = srcc, dstc
        else:
            nch = nchunk
            sbuf, dbuf = srcf, dstf

        plsc.subcore_barrier()

        half = CH // 2

        def cbody(j, carry):
            for t in range(CH // LANES):
                srcix[pl.ds(t * LANES, LANES)] = (
                    sbuf[pl.ds(j * CH + t * LANES, LANES)])
            for t in range(half // LANES):
                dstix0[pl.ds(t * LANES, LANES)] = (
                    dbuf[pl.ds(j * CH + t * LANES, LANES)])
                dstix1[pl.ds(t * LANES, LANES)] = (
                    dbuf[pl.ds(j * CH + half + t * LANES, LANES)])
            c0 = pltpu.async_copy(
                tab.at[srcix.at[pl.ds(0, half)]], rows0, gsem0)
            c1 = pltpu.async_copy(
                tab.at[srcix.at[pl.ds(half, half)]], rows1, gsem1)
            c0.wait()
            pltpu.sync_copy(rows0, acc.at[dstix0], add=True)
            pltpu.sync_copy(ones, cnt.at[dstix0], add=True)
            c1.wait()
            pltpu.sync_copy(rows1, acc.at[dstix1], add=True)
            pltpu.sync_copy(ones, cnt.at[dstix1], add=True)
            return carry
        lax.fori_loop(0, nch, cbody, 0)

        plsc.subcore_barrier()

        pltpu.sync_copy(acc.at[pl.ds(rbase, ROWS_PER_TILE)],
                        out_acc.at[c].at[pl.ds(rbase, ROWS_PER_TILE)])
        pltpu.sync_copy(cnt.at[pl.ds(rbase, ROWS_PER_TILE)],
                        out_cnt.at[c].at[pl.ds(rbase, ROWS_PER_TILE)])

    return agg


_aggs = {}


def _get_agg(ept, remap, stage_table=False):
    key = (ept, remap, stage_table)
    if key not in _aggs:
        _aggs[key] = _make_agg(ept, remap, stage_table)
    return _aggs[key]


def _dense_body(accp, cntp, xt, wl, bl, wr, out, *, sig):
    a = accp[0, :N2, :] + accp[1, :N2, :]
    ct = cntp[0, :N2, 0] + cntp[1, :N2, 0]
    inv = 1.0 / jnp.maximum(ct, 1.0)
    mean = a * inv[:, None]
    r = (jnp.dot(mean, wl[...], preferred_element_type=jnp.float32)
         + jnp.dot(xt[...], wr[...], preferred_element_type=jnp.float32)
         + bl[...])
    if sig:
        r = 1.0 / (1.0 + jnp.exp(-r))
    out[...] = r


def _dense(accp, cntp, xt, wl, bl, wr, sig):
    return pl.pallas_call(
        functools.partial(_dense_body, sig=sig),
        out_shape=jax.ShapeDtypeStruct((N2, D), jnp.float32),
    )(accp, cntp, xt, wl, bl, wr)


def kernel(x, edge_index1, edge_index2, size1_dst, size2_dst,
           W_l1, b_l1, W_r1, W_l2, b_l2, W_r2):
    e1 = edge_index1.shape[1]
    e2 = edge_index2.shape[1]

    agg1 = _get_agg(e1 // NW, True)
    agg2 = _get_agg(e2 // NW, False, stage_table=True)

    zrow = jnp.zeros((ACC_ROWS, D), jnp.float32)
    zcnt = jnp.zeros((ACC_ROWS, CW), jnp.float32)
    onesh = jnp.ones((CH // 2, CW), jnp.float32)

    accp1, cntp1 = agg1(x, edge_index1.reshape(2 * e1), zrow, zcnt, onesh)
    h = _dense(accp1, cntp1, x[:N2], W_l1, b_l1.reshape(1, D), W_r1, True)
    accp2, cntp2 = agg2(h, edge_index2.reshape(2 * e2), zrow, zcnt, onesh)
    out = _dense(accp2, cntp2, h, W_l2, b_l2.reshape(1, D), W_r2, False)
    return out
